# baseline (speedup 1.0000x reference)
import jax
import jax.numpy as jnp
from jax.experimental import pallas as pl
from jax.experimental.pallas import tpu as pltpu

_BS = 5
_ST = 16
_CTA = 128
_CTC = 128
_PCORES = 2
_APRON = 8
_IMIN = -2147483648


def _window_count(gamma, seed_idx, band, scr, h, w):
    ct = _ST
    hv = h - (_BS - 1)
    pltpu.prng_seed(seed_idx)
    bits = pltpu.bitcast(pltpu.prng_random_bits((ct, hv, w)), jnp.int32)
    thresh = (float(_IMIN) + jnp.clip(gamma, 0.0, 1.0) * 4294967296.0
              ).astype(jnp.int32)
    lane = jax.lax.broadcasted_iota(jnp.int32, (1, 1, w), 2)
    tvec = jnp.where(lane < (w - (_BS - 1)), thresh, jnp.int32(_IMIN))
    m = jnp.where(bits < tvec, 1.0, 0.0).astype(jnp.bfloat16)
    scr[:, 0:_APRON, :] = jnp.zeros((ct, _APRON, w), jnp.bfloat16)
    scr[:, _APRON:_APRON + hv, :] = m
    scr[:, _APRON + hv:, :] = jnp.zeros((ct, h - hv, w), jnp.bfloat16)
    r = jnp.maximum(
        jnp.maximum(
            jnp.maximum(scr[:, _APRON:_APRON + h, :],
                        scr[:, _APRON - 1:_APRON - 1 + h, :]),
            jnp.maximum(scr[:, _APRON - 2:_APRON - 2 + h, :],
                        scr[:, _APRON - 3:_APRON - 3 + h, :])),
        scr[:, _APRON - 4:_APRON - 4 + h, :])
    cnt = jax.lax.dot_general(
        r.reshape(ct * h, w), band,
        (((1,), (0,)), ((), ())), preferred_element_type=jnp.float32)
    return cnt.reshape(ct, h, w)


def _count_body(h, w, inner):
    tiles = _CTC // _ST

    def body(gamma_ref, band_ref, out_ref, acc_ref, scr_ref, scr2_ref):
        p = pl.program_id(0)
        s = pl.program_id(1)
        band = band_ref[...]
        part = 0.0
        for j in range(tiles):
            cnt = _window_count(gamma_ref[0, 0], (p * inner + s) * tiles + j,
                                band, scr_ref if j % 2 == 0 else scr2_ref,
                                h, w)
            part += jnp.sum(jnp.minimum(cnt, 1.0))

        @pl.when(s == 0)
        def _():
            acc_ref[0, 0] = 0.0

        acc_ref[0, 0] += part

        @pl.when(s == inner - 1)
        def _():
            out_ref[p, 0] = acc_ref[0, 0]

    return body


def _apply_body(h, w, count_m):
    tiles = _CTA // _ST

    def body(gamma_ref, cnt_ref, band_ref, x_ref, out_ref, scr_ref, scr2_ref):
        i = pl.program_id(0)
        band = band_ref[...]
        sum_dilated = cnt_ref[0, 0] + cnt_ref[1, 0]
        scale = count_m / (count_m - sum_dilated)
        for j in range(tiles):
            cnt = _window_count(gamma_ref[0, 0], i * tiles + j, band, h=h,
                                w=w, scr=scr_ref if j % 2 == 0 else scr2_ref)
            sl = pl.ds(j * _ST, _ST)
            out_ref[sl, :, :] = jnp.where(cnt > 0.5, 0.0,
                                          x_ref[sl, :, :] * scale)

    return body


def kernel(x, gamma):
    b, c, h, w = x.shape
    n = b * c
    inner = n // _CTC // _PCORES
    xf = x.reshape(n, h, w)
    g = gamma.reshape(1, 1).astype(jnp.float32)
    count_m = float(b * c * h * w)
    diff = jnp.arange(w)[None, :] - jnp.arange(w)[:, None]
    band = ((diff >= 0) & (diff < _BS)).astype(jnp.bfloat16)
    scr = pltpu.VMEM((_ST, h + _APRON, w), jnp.bfloat16)

    partials = pl.pallas_call(
        _count_body(h, w, inner),
        grid=(_PCORES, inner),
        in_specs=[
            pl.BlockSpec(memory_space=pltpu.SMEM),
            pl.BlockSpec((w, w), lambda p, s: (0, 0)),
        ],
        out_specs=pl.BlockSpec(memory_space=pltpu.SMEM),
        out_shape=jax.ShapeDtypeStruct((_PCORES, 1), jnp.float32),
        scratch_shapes=[pltpu.SMEM((1, 1), jnp.float32), scr, scr],
        compiler_params=pltpu.CompilerParams(
            dimension_semantics=("parallel", "arbitrary"),
        ),
    )(g, band)

    out = pl.pallas_call(
        _apply_body(h, w, count_m),
        grid=(n // _CTA,),
        in_specs=[
            pl.BlockSpec(memory_space=pltpu.SMEM),
            pl.BlockSpec(memory_space=pltpu.SMEM),
            pl.BlockSpec((w, w), lambda i: (0, 0)),
            pl.BlockSpec((_CTA, h, w), lambda i: (i, 0, 0)),
        ],
        out_specs=pl.BlockSpec((_CTA, h, w), lambda i: (i, 0, 0)),
        out_shape=jax.ShapeDtypeStruct((n, h, w), jnp.float32),
        scratch_shapes=[scr, scr],
        compiler_params=pltpu.CompilerParams(
            dimension_semantics=("parallel",),
        ),
    )(g, partials, band, xf)

    return out.reshape(b, c, h, w)


# speedup vs baseline: 1.6508x; 1.6508x over previous
import jax
import jax.numpy as jnp
from jax.experimental import pallas as pl
from jax.experimental.pallas import tpu as pltpu

_BS = 5
_ST = 16
_CTA = 128
_CTC = 128
_PCORES = 2
_APRON = 8
_IMIN = -2147483648


def _window_count(gamma, seed_idx, band, scr, h, w):
    ct = _ST
    hv = h - (_BS - 1)
    pltpu.prng_seed(seed_idx)
    bits = pltpu.bitcast(pltpu.prng_random_bits((ct, hv, w)), jnp.int32)
    thresh = (float(_IMIN) + jnp.clip(gamma, 0.0, 1.0) * 4294967296.0
              ).astype(jnp.int32)
    lane = jax.lax.broadcasted_iota(jnp.int32, (1, 1, w), 2)
    tvec = jnp.where(lane < (w - (_BS - 1)), thresh, jnp.int32(_IMIN))
    m = jnp.where(bits < tvec, 1.0, 0.0)
    scr[:, 0:_APRON, :] = jnp.zeros((ct, _APRON, w), jnp.float32)
    scr[:, _APRON:_APRON + hv, :] = m
    scr[:, _APRON + hv:, :] = jnp.zeros((ct, h - hv, w), jnp.float32)
    r = jnp.maximum(
        jnp.maximum(
            jnp.maximum(scr[:, _APRON:_APRON + h, :],
                        scr[:, _APRON - 1:_APRON - 1 + h, :]),
            jnp.maximum(scr[:, _APRON - 2:_APRON - 2 + h, :],
                        scr[:, _APRON - 3:_APRON - 3 + h, :])),
        scr[:, _APRON - 4:_APRON - 4 + h, :])
    cnt = jax.lax.dot_general(
        r.astype(jnp.bfloat16).reshape(ct * h, w), band,
        (((1,), (0,)), ((), ())), preferred_element_type=jnp.float32)
    return cnt.reshape(ct, h, w)


def _fused_body(h, w, steps, count_m):
    tiles = _CTA // _ST

    def body(gamma_ref, band_ref, x_ref, out_ref, acc_ref, scr_ref, scr2_ref):
        s = pl.program_id(0)
        band = band_ref[...]

        @pl.when(s == 0)
        def _():
            acc_ref[0, 0] = 0.0

        @pl.when(s < steps)
        def _count_phase():
            part = 0.0
            for j in range(tiles):
                cnt = _window_count(gamma_ref[0, 0], s * tiles + j,
                                    band, scr_ref if j % 2 == 0 else scr2_ref,
                                    h, w)
                part += jnp.sum(jnp.minimum(cnt, 1.0))
            acc_ref[0, 0] += part

        @pl.when(s >= steps)
        def _apply_phase():
            scale = count_m / (count_m - acc_ref[0, 0])
            for j in range(tiles):
                cnt = _window_count(gamma_ref[0, 0], (s - steps) * tiles + j,
                                    band, scr_ref if j % 2 == 0 else scr2_ref,
                                    h, w)
                sl = pl.ds(j * _ST, _ST)
                out_ref[sl, :, :] = jnp.where(cnt > 0.5, 0.0,
                                              x_ref[sl, :, :] * scale)

    return body


def kernel(x, gamma):
    b, c, h, w = x.shape
    n = b * c
    steps = n // _CTA
    xf = x.reshape(n, h, w)
    g = gamma.reshape(1, 1).astype(jnp.float32)
    count_m = float(b * c * h * w)
    diff = jnp.arange(w)[None, :] - jnp.arange(w)[:, None]
    band = ((diff >= 0) & (diff < _BS)).astype(jnp.bfloat16)
    scr = pltpu.VMEM((_ST, h + _APRON, w), jnp.float32)

    out = pl.pallas_call(
        _fused_body(h, w, steps, count_m),
        grid=(2 * steps,),
        in_specs=[
            pl.BlockSpec(memory_space=pltpu.SMEM),
            pl.BlockSpec((w, w), lambda s: (0, 0)),
            pl.BlockSpec((_CTA, h, w), lambda s: (jnp.maximum(s - steps, 0), 0, 0)),
        ],
        out_specs=pl.BlockSpec((_CTA, h, w), lambda s: (jnp.maximum(s - steps, 0), 0, 0)),
        out_shape=jax.ShapeDtypeStruct((n, h, w), jnp.float32),
        scratch_shapes=[pltpu.SMEM((1, 1), jnp.float32), scr, scr],
        compiler_params=pltpu.CompilerParams(
            dimension_semantics=("arbitrary",),
        ),
    )(g, band, xf)

    return out.reshape(b, c, h, w)


# speedup vs baseline: 1.6546x; 1.0023x over previous
import jax
import jax.numpy as jnp
from jax.experimental import pallas as pl
from jax.experimental.pallas import tpu as pltpu

_BS = 5
_ST = 16
_CTA = 192
_CTC = 128
_PCORES = 2
_APRON = 8
_IMIN = -2147483648


def _window_count(gamma, seed_idx, band, scr, h, w):
    ct = _ST
    hv = h - (_BS - 1)
    pltpu.prng_seed(seed_idx)
    bits = pltpu.bitcast(pltpu.prng_random_bits((ct, hv, w)), jnp.int32)
    thresh = (float(_IMIN) + jnp.clip(gamma, 0.0, 1.0) * 4294967296.0
              ).astype(jnp.int32)
    lane = jax.lax.broadcasted_iota(jnp.int32, (1, 1, w), 2)
    tvec = jnp.where(lane < (w - (_BS - 1)), thresh, jnp.int32(_IMIN))
    m = jnp.where(bits < tvec, 1.0, 0.0)
    scr[:, 0:_APRON, :] = jnp.zeros((ct, _APRON, w), jnp.float32)
    scr[:, _APRON:_APRON + hv, :] = m
    scr[:, _APRON + hv:, :] = jnp.zeros((ct, h - hv, w), jnp.float32)
    r = jnp.maximum(
        jnp.maximum(
            jnp.maximum(scr[:, _APRON:_APRON + h, :],
                        scr[:, _APRON - 1:_APRON - 1 + h, :]),
            jnp.maximum(scr[:, _APRON - 2:_APRON - 2 + h, :],
                        scr[:, _APRON - 3:_APRON - 3 + h, :])),
        scr[:, _APRON - 4:_APRON - 4 + h, :])
    cnt = jax.lax.dot_general(
        r.astype(jnp.bfloat16).reshape(ct * h, w), band,
        (((1,), (0,)), ((), ())), preferred_element_type=jnp.float32)
    return cnt.reshape(ct, h, w)


def _fused_body(h, w, steps, count_m):
    tiles = _CTA // _ST

    def body(gamma_ref, band_ref, x_ref, out_ref, acc_ref, scr_ref, scr2_ref):
        s = pl.program_id(0)
        band = band_ref[...]

        @pl.when(s == 0)
        def _():
            acc_ref[0, 0] = 0.0

        @pl.when(s < steps)
        def _count_phase():
            part = 0.0
            for j in range(tiles):
                cnt = _window_count(gamma_ref[0, 0], s * tiles + j,
                                    band, scr_ref if j % 2 == 0 else scr2_ref,
                                    h, w)
                part += jnp.sum(jnp.minimum(cnt, 1.0))
            acc_ref[0, 0] += part

        @pl.when(s >= steps)
        def _apply_phase():
            scale = count_m / (count_m - acc_ref[0, 0])
            for j in range(tiles):
                cnt = _window_count(gamma_ref[0, 0], (s - steps) * tiles + j,
                                    band, scr_ref if j % 2 == 0 else scr2_ref,
                                    h, w)
                sl = pl.ds(j * _ST, _ST)
                out_ref[sl, :, :] = jnp.where(cnt > 0.5, 0.0,
                                              x_ref[sl, :, :] * scale)

    return body


def kernel(x, gamma):
    b, c, h, w = x.shape
    n = b * c
    steps = n // _CTA
    xf = x.reshape(n, h, w)
    g = gamma.reshape(1, 1).astype(jnp.float32)
    count_m = float(b * c * h * w)
    diff = jnp.arange(w)[None, :] - jnp.arange(w)[:, None]
    band = ((diff >= 0) & (diff < _BS)).astype(jnp.bfloat16)
    scr = pltpu.VMEM((_ST, h + _APRON, w), jnp.float32)

    out = pl.pallas_call(
        _fused_body(h, w, steps, count_m),
        grid=(2 * steps,),
        in_specs=[
            pl.BlockSpec(memory_space=pltpu.SMEM),
            pl.BlockSpec((w, w), lambda s: (0, 0)),
            pl.BlockSpec((_CTA, h, w), lambda s: (jnp.maximum(s - steps, 0), 0, 0)),
        ],
        out_specs=pl.BlockSpec((_CTA, h, w), lambda s: (jnp.maximum(s - steps, 0), 0, 0)),
        out_shape=jax.ShapeDtypeStruct((n, h, w), jnp.float32),
        scratch_shapes=[pltpu.SMEM((1, 1), jnp.float32), scr, scr],
        compiler_params=pltpu.CompilerParams(
            dimension_semantics=("arbitrary",),
        ),
    )(g, band, xf)

    return out.reshape(b, c, h, w)


# speedup vs baseline: 1.6665x; 1.0072x over previous
import jax
import jax.numpy as jnp
from jax.experimental import pallas as pl
from jax.experimental.pallas import tpu as pltpu

_BS = 5
_ST = 16
_CTA = 192
_CTC = 128
_PCORES = 2
_APRON = 8
_IMIN = -2147483648


def _window_count(gamma, seed_idx, band, scr, h, w):
    ct = _ST
    hv = h - (_BS - 1)
    pltpu.prng_seed(seed_idx)
    bits = pltpu.bitcast(pltpu.prng_random_bits((ct, hv, w)), jnp.int32)
    thresh = (float(_IMIN) + jnp.clip(gamma, 0.0, 1.0) * 4294967296.0
              ).astype(jnp.int32)
    lane = jax.lax.broadcasted_iota(jnp.int32, (1, 1, w), 2)
    tvec = jnp.where(lane < (w - (_BS - 1)), thresh, jnp.int32(_IMIN))
    m = jnp.where(bits < tvec, 1.0, 0.0)
    scr[:, 0:_APRON, :] = jnp.zeros((ct, _APRON, w), jnp.float32)
    scr[:, _APRON:_APRON + hv, :] = m
    scr[:, _APRON + hv:, :] = jnp.zeros((ct, h - hv, w), jnp.float32)
    r = jnp.maximum(
        jnp.maximum(
            jnp.maximum(scr[:, _APRON:_APRON + h, :],
                        scr[:, _APRON - 1:_APRON - 1 + h, :]),
            jnp.maximum(scr[:, _APRON - 2:_APRON - 2 + h, :],
                        scr[:, _APRON - 3:_APRON - 3 + h, :])),
        scr[:, _APRON - 4:_APRON - 4 + h, :])
    cnt = jax.lax.dot_general(
        r.astype(jnp.bfloat16).reshape(ct * h, w), band,
        (((1,), (0,)), ((), ())), preferred_element_type=jnp.float32)
    return cnt.reshape(ct, h, w)


def _fused_body(h, w, steps, count_m):
    tiles = _CTA // _ST

    def body(gamma_ref, band_ref, x_ref, out_ref, acc_ref, scr_ref, scr2_ref):
        s = pl.program_id(0)
        band = band_ref[...]

        @pl.when(s == 0)
        def _():
            acc_ref[0, 0] = 0.0

        @pl.when(s < steps)
        def _count_phase():
            part_vec = jnp.zeros((_ST, h, w), jnp.float32)
            for j in range(tiles):
                cnt = _window_count(gamma_ref[0, 0], s * tiles + j,
                                    band, scr_ref if j % 2 == 0 else scr2_ref,
                                    h, w)
                part_vec += jnp.minimum(cnt, 1.0)
            acc_ref[0, 0] += jnp.sum(part_vec)

        @pl.when(s >= steps)
        def _apply_phase():
            scale = count_m / (count_m - acc_ref[0, 0])
            for j in range(tiles):
                cnt = _window_count(gamma_ref[0, 0], (s - steps) * tiles + j,
                                    band, scr_ref if j % 2 == 0 else scr2_ref,
                                    h, w)
                sl = pl.ds(j * _ST, _ST)
                out_ref[sl, :, :] = jnp.where(cnt > 0.5, 0.0,
                                              x_ref[sl, :, :] * scale)

    return body


def kernel(x, gamma):
    b, c, h, w = x.shape
    n = b * c
    steps = n // _CTA
    xf = x.reshape(n, h, w)
    g = gamma.reshape(1, 1).astype(jnp.float32)
    count_m = float(b * c * h * w)
    diff = jnp.arange(w)[None, :] - jnp.arange(w)[:, None]
    band = ((diff >= 0) & (diff < _BS)).astype(jnp.bfloat16)
    scr = pltpu.VMEM((_ST, h + _APRON, w), jnp.float32)

    out = pl.pallas_call(
        _fused_body(h, w, steps, count_m),
        grid=(2 * steps,),
        in_specs=[
            pl.BlockSpec(memory_space=pltpu.SMEM),
            pl.BlockSpec((w, w), lambda s: (0, 0)),
            pl.BlockSpec((_CTA, h, w), lambda s: (jnp.maximum(s - steps, 0), 0, 0)),
        ],
        out_specs=pl.BlockSpec((_CTA, h, w), lambda s: (jnp.maximum(s - steps, 0), 0, 0)),
        out_shape=jax.ShapeDtypeStruct((n, h, w), jnp.float32),
        scratch_shapes=[pltpu.SMEM((1, 1), jnp.float32), scr, scr],
        compiler_params=pltpu.CompilerParams(
            dimension_semantics=("arbitrary",),
        ),
    )(g, band, xf)

    return out.reshape(b, c, h, w)


# speedup vs baseline: 1.6751x; 1.0052x over previous
import jax
import jax.numpy as jnp
from jax.experimental import pallas as pl
from jax.experimental.pallas import tpu as pltpu

_BS = 5
_ST = 16
_CTA = 192
_CTC = 128
_PCORES = 2
_APRON = 8
_IMIN = -2147483648


def _lane_thresholds(gamma, w):
    thresh = (float(_IMIN) + jnp.clip(gamma, 0.0, 1.0) * 4294967296.0
              ).astype(jnp.int32)
    lane = jax.lax.broadcasted_iota(jnp.int32, (1, 1, w), 2)
    return jnp.where(lane < (w - (_BS - 1)), thresh, jnp.int32(_IMIN))


def _init_aprons(scr, h, w):
    hv = h - (_BS - 1)
    scr[:, 0:_APRON, :] = jnp.zeros((_ST, _APRON, w), jnp.float32)
    scr[:, _APRON + hv:, :] = jnp.zeros((_ST, h - hv, w), jnp.float32)


def _window_count(tvec, seed_idx, band, scr, h, w):
    ct = _ST
    hv = h - (_BS - 1)
    pltpu.prng_seed(seed_idx)
    bits = pltpu.bitcast(pltpu.prng_random_bits((ct, hv, w)), jnp.int32)
    m = jnp.where(bits < tvec, 1.0, 0.0)
    scr[:, _APRON:_APRON + hv, :] = m
    r = jnp.maximum(
        jnp.maximum(
            jnp.maximum(scr[:, _APRON:_APRON + h, :],
                        scr[:, _APRON - 1:_APRON - 1 + h, :]),
            jnp.maximum(scr[:, _APRON - 2:_APRON - 2 + h, :],
                        scr[:, _APRON - 3:_APRON - 3 + h, :])),
        scr[:, _APRON - 4:_APRON - 4 + h, :])
    cnt = jax.lax.dot_general(
        r.astype(jnp.bfloat16).reshape(ct * h, w), band,
        (((1,), (0,)), ((), ())), preferred_element_type=jnp.float32)
    return cnt.reshape(ct, h, w)


def _fused_body(h, w, steps, count_m):
    tiles = _CTA // _ST

    def body(gamma_ref, band_ref, x_ref, out_ref, acc_ref, scr_ref, scr2_ref):
        s = pl.program_id(0)
        band = band_ref[...]
        tvec = _lane_thresholds(gamma_ref[0, 0], w)

        @pl.when(s == 0)
        def _():
            acc_ref[0, 0] = 0.0
            _init_aprons(scr_ref, h, w)
            _init_aprons(scr2_ref, h, w)

        @pl.when(s < steps)
        def _count_phase():
            part_vec = jnp.zeros((_ST, h, w), jnp.float32)
            for j in range(tiles):
                cnt = _window_count(tvec, s * tiles + j,
                                    band, scr_ref if j % 2 == 0 else scr2_ref,
                                    h, w)
                part_vec += jnp.minimum(cnt, 1.0)
            acc_ref[0, 0] += jnp.sum(part_vec)

        @pl.when(s >= steps)
        def _apply_phase():
            scale = count_m / (count_m - acc_ref[0, 0])
            for j in range(tiles):
                cnt = _window_count(tvec, (s - steps) * tiles + j,
                                    band, scr_ref if j % 2 == 0 else scr2_ref,
                                    h, w)
                sl = pl.ds(j * _ST, _ST)
                out_ref[sl, :, :] = jnp.where(cnt > 0.5, 0.0,
                                              x_ref[sl, :, :] * scale)

    return body


def kernel(x, gamma):
    b, c, h, w = x.shape
    n = b * c
    steps = n // _CTA
    xf = x.reshape(n, h, w)
    g = gamma.reshape(1, 1).astype(jnp.float32)
    count_m = float(b * c * h * w)
    diff = jnp.arange(w)[None, :] - jnp.arange(w)[:, None]
    band = ((diff >= 0) & (diff < _BS)).astype(jnp.bfloat16)
    scr = pltpu.VMEM((_ST, h + _APRON, w), jnp.float32)

    out = pl.pallas_call(
        _fused_body(h, w, steps, count_m),
        grid=(2 * steps,),
        in_specs=[
            pl.BlockSpec(memory_space=pltpu.SMEM),
            pl.BlockSpec((w, w), lambda s: (0, 0)),
            pl.BlockSpec((_CTA, h, w), lambda s: (jnp.maximum(s - steps, 0), 0, 0)),
        ],
        out_specs=pl.BlockSpec((_CTA, h, w), lambda s: (jnp.maximum(s - steps, 0), 0, 0)),
        out_shape=jax.ShapeDtypeStruct((n, h, w), jnp.float32),
        scratch_shapes=[pltpu.SMEM((1, 1), jnp.float32), scr, scr],
        compiler_params=pltpu.CompilerParams(
            dimension_semantics=("arbitrary",),
        ),
    )(g, band, xf)

    return out.reshape(b, c, h, w)


# speedup vs baseline: 1.6779x; 1.0017x over previous
import jax
import jax.numpy as jnp
from jax.experimental import pallas as pl
from jax.experimental.pallas import tpu as pltpu

_BS = 5
_ST = 16
_CTA = 192
_APRON = 8
_IMIN = -2147483648


def _lane_thresholds(gamma, w):
    thresh = (float(_IMIN) + jnp.clip(gamma, 0.0, 1.0) * 4294967296.0
              ).astype(jnp.int32)
    lane = jax.lax.broadcasted_iota(jnp.int32, (1, 1, w), 2)
    return jnp.where(lane < (w - (_BS - 1)), thresh, jnp.int32(_IMIN))


def _init_aprons(scr, h, w):
    hv = h - (_BS - 1)
    scr[:, 0:_APRON, :] = jnp.zeros((_ST, _APRON, w), jnp.float32)
    scr[:, _APRON + hv:, :] = jnp.zeros((_ST, h - hv, w), jnp.float32)


def _window_count(tvec, seed_idx, band, scr, h, w):
    ct = _ST
    hv = h - (_BS - 1)
    pltpu.prng_seed(seed_idx)
    bits = pltpu.bitcast(pltpu.prng_random_bits((ct, hv, w)), jnp.int32)
    m = jnp.where(bits < tvec, 1.0, 0.0)
    scr[:, _APRON:_APRON + hv, :] = m
    r = jnp.maximum(
        jnp.maximum(
            jnp.maximum(scr[:, _APRON:_APRON + h, :],
                        scr[:, _APRON - 1:_APRON - 1 + h, :]),
            jnp.maximum(scr[:, _APRON - 2:_APRON - 2 + h, :],
                        scr[:, _APRON - 3:_APRON - 3 + h, :])),
        scr[:, _APRON - 4:_APRON - 4 + h, :])
    cnt = jax.lax.dot_general(
        r.astype(jnp.bfloat16).reshape(ct * h, w), band,
        (((1,), (0,)), ((), ())), preferred_element_type=jnp.float32)
    return cnt.reshape(ct, h, w)


def _fused_body(h, w, steps, count_m):
    tiles = _CTA // _ST

    def body(gamma_ref, band_ref, x_ref, out_ref, acc_ref, scr_ref, scr2_ref):
        s = pl.program_id(0)
        band = band_ref[...]
        tvec = _lane_thresholds(gamma_ref[0, 0], w)

        @pl.when(s == 0)
        def _():
            acc_ref[0, 0] = 0.0
            _init_aprons(scr_ref, h, w)
            _init_aprons(scr2_ref, h, w)

        @pl.when(s < steps)
        def _count_phase():
            part_vec = jnp.zeros((_ST, h, w), jnp.float32)
            for j in range(tiles):
                cnt = _window_count(tvec, s * tiles + j,
                                    band, scr_ref if j % 2 == 0 else scr2_ref,
                                    h, w)
                part_vec += jnp.minimum(cnt, 1.0)
            acc_ref[0, 0] += jnp.sum(part_vec)

        @pl.when(s >= steps)
        def _apply_phase():
            scale = count_m / (count_m - acc_ref[0, 0])
            for j in range(tiles):
                cnt = _window_count(tvec, (s - steps) * tiles + j,
                                    band, scr_ref if j % 2 == 0 else scr2_ref,
                                    h, w)
                sl = pl.ds(j * _ST, _ST)
                out_ref[sl, :, :] = jnp.where(cnt > 0.5, 0.0,
                                              x_ref[sl, :, :] * scale)

    return body


def kernel(x, gamma):
    b, c, h, w = x.shape
    n = b * c
    steps = n // _CTA
    xf = x.reshape(n, h, w)
    g = gamma.reshape(1, 1).astype(jnp.float32)
    count_m = float(b * c * h * w)
    diff = jnp.arange(w)[None, :] - jnp.arange(w)[:, None]
    band = ((diff >= 0) & (diff < _BS)).astype(jnp.bfloat16)
    scr = pltpu.VMEM((_ST, h + _APRON, w), jnp.float32)

    out = pl.pallas_call(
        _fused_body(h, w, steps, count_m),
        grid=(2 * steps,),
        in_specs=[
            pl.BlockSpec(memory_space=pltpu.SMEM),
            pl.BlockSpec((w, w), lambda s: (0, 0)),
            pl.BlockSpec((_CTA, h, w), lambda s: (jnp.maximum(s - steps, 0), 0, 0)),
        ],
        out_specs=pl.BlockSpec((_CTA, h, w), lambda s: (jnp.maximum(s - steps, 0), 0, 0)),
        out_shape=jax.ShapeDtypeStruct((n, h, w), jnp.float32),
        scratch_shapes=[pltpu.SMEM((1, 1), jnp.float32), scr, scr],
        compiler_params=pltpu.CompilerParams(
            dimension_semantics=("arbitrary",),
        ),
    )(g, band, xf)

    return out.reshape(b, c, h, w)
